# baseline (speedup 1.0000x reference)
import functools

import jax
import jax.numpy as jnp
from jax import lax
from jax.experimental import pallas as pl
from jax.experimental.pallas import tpu as pltpu
from jax.experimental.pallas import tpu_sc as plsc

N = 2048
ROW = 32512
OFFSET = 32000
EXTRA = ROW - OFFSET
EPS = 1e-20
UNK = 0
IGNORE_INDEX = -100

NC = 2
NS = 16
NW = NC * NS
RPW = N // NW
L = 16
CH = RPW // L
BLK = RPW // 8

_LN2 = 0.6931471805599453
_SQRTHF = 0.70710678118654752440


def _vlog(x):
    bits = lax.bitcast_convert_type(x, jnp.int32)
    e = lax.shift_right_arithmetic(bits, 23) - 126
    m_bits = (bits & 0x007FFFFF) | 0x3F000000
    m = lax.bitcast_convert_type(m_bits, jnp.float32)
    small = m < _SQRTHF
    e = jnp.where(small, e - 1, e).astype(jnp.float32)
    t = jnp.where(small, m + m, m) - 1.0
    p = jnp.float32(7.0376836292e-2)
    p = p * t + jnp.float32(-1.1514610310e-1)
    p = p * t + jnp.float32(1.1676998740e-1)
    p = p * t + jnp.float32(-1.2420140846e-1)
    p = p * t + jnp.float32(1.4249322787e-1)
    p = p * t + jnp.float32(-1.6668057665e-1)
    p = p * t + jnp.float32(2.0000714765e-1)
    p = p * t + jnp.float32(-2.4999993993e-1)
    p = p * t + jnp.float32(3.3333331174e-1)
    t2 = t * t
    y = t2 * (t * p - 0.5)
    return t + y + e * jnp.float32(_LN2)


def _make_half_kernel(row0: int, nrows: int):
    rpw = nrows // NW
    ch = rpw // L
    blk = rpw // 8

    @functools.partial(
        pl.kernel,
        mesh=plsc.VectorSubcoreMesh(core_axis_name="c", subcore_axis_name="s"),
        out_type=jax.ShapeDtypeStruct((nrows,), jnp.float32),
        compiler_params=pltpu.CompilerParams(
            needs_layout_passes=False, disable_bounds_checks=True,
            disable_semaphore_checks=True),
        scratch_types=[
            pltpu.VMEM((rpw,), jnp.int32),
            pltpu.VMEM((rpw,), jnp.int32),
            pltpu.VMEM((blk, 8, EXTRA), jnp.float32),
            pltpu.VMEM((rpw, 8, 128), jnp.float32),
            pltpu.VMEM((rpw,), jnp.float32),
            pltpu.SemaphoreType.DMA,
        ],
    )
    def _loss_kernel(scores_hbm, align_hbm, target_hbm, out_hbm,
                     align_v, target_v, slab_a, blk_t, out_v, sem):
        wid = lax.axis_index("s") * NC + lax.axis_index("c")
        obase = wid * rpw
        base = row0 + obase
        cp_a = pltpu.async_copy(align_hbm.at[pl.ds(obase, rpw)], align_v, sem)
        cp_t = pltpu.async_copy(target_hbm.at[pl.ds(obase, rpw)], target_v, sem)
        copies = []
        for b in range(blk):
            copies.append(pltpu.async_copy(
                scores_hbm.at[pl.ds(pl.multiple_of(base + b * 8, 8), 8),
                              pl.ds(OFFSET, EXTRA)],
                slab_a.at[b], sem))
        cp_a.wait()
        cp_t.wait()
        lanes = lax.iota(jnp.int32, 16)
        for c in range(ch):
            tv = target_v[pl.ds(c * L, L)]
            for j in range(L):
                r = c * L + j
                t_scal = jnp.max(jnp.where(lanes == j, tv, 0))
                t_tile = pl.multiple_of((t_scal >> 7) << 7, 128)
                copies.append(pltpu.async_copy(
                    scores_hbm.at[pl.ds(pl.multiple_of(base + (r & ~7), 8), 8),
                                  pl.ds(t_tile, 128)],
                    blk_t.at[r], sem))
        for cp in copies:
            cp.wait()
        for c in range(ch):
            av = align_v[pl.ds(c * L, L)]
            tv = target_v[pl.ds(c * L, L)]
            r_vec = c * L + lanes
            sub = r_vec & 7
            a_val = plsc.load_gather(slab_a, [r_vec >> 3, sub, av])
            t_val = plsc.load_gather(blk_t, [r_vec, sub, tv & 127])
            zero = jnp.zeros((L,), jnp.float32)
            a_unk = av == UNK
            t_unk = tv == UNK
            out = jnp.where(a_unk, zero, a_val) + jnp.float32(EPS)
            out = out + jnp.where(t_unk, zero, t_val)
            out = out + jnp.where(a_unk & t_unk, t_val, zero)
            loss = -_vlog(out)
            loss = jnp.where(tv == IGNORE_INDEX, zero, loss)
            out_v[pl.ds(c * L, L)] = loss
        pltpu.sync_copy(out_v, out_hbm.at[pl.ds(obase, rpw)])

    return _loss_kernel


_half0 = _make_half_kernel(0, N // 2)
_half1 = _make_half_kernel(N // 2, N // 2)


def kernel(scores, align, target):
    align = align.astype(jnp.int32)
    target = target.astype(jnp.int32)
    h = N // 2
    lo = _half0(scores, align[:h], target[:h])
    hi = _half1(scores, align[h:], target[h:])
    return jnp.concatenate([lo, hi])


# speedup vs baseline: 1.2319x; 1.2319x over previous
import functools

import jax
import jax.numpy as jnp
from jax import lax
from jax.experimental import pallas as pl
from jax.experimental.pallas import tpu as pltpu
from jax.experimental.pallas import tpu_sc as plsc

N = 2048
ROW = 32512
OFFSET = 32000
EXTRA = ROW - OFFSET
EPS = 1e-20
UNK = 0
IGNORE_INDEX = -100

NC = 2
NS = 16
NW = NC * NS
RPW = N // NW
L = 16
CH = RPW // L
BLK = RPW // 8

_LN2 = 0.6931471805599453
_SQRTHF = 0.70710678118654752440


def _vlog(x):
    bits = lax.bitcast_convert_type(x, jnp.int32)
    e = lax.shift_right_arithmetic(bits, 23) - 126
    m_bits = (bits & 0x007FFFFF) | 0x3F000000
    m = lax.bitcast_convert_type(m_bits, jnp.float32)
    small = m < _SQRTHF
    e = jnp.where(small, e - 1, e).astype(jnp.float32)
    t = jnp.where(small, m + m, m) - 1.0
    p = jnp.float32(7.0376836292e-2)
    p = p * t + jnp.float32(-1.1514610310e-1)
    p = p * t + jnp.float32(1.1676998740e-1)
    p = p * t + jnp.float32(-1.2420140846e-1)
    p = p * t + jnp.float32(1.4249322787e-1)
    p = p * t + jnp.float32(-1.6668057665e-1)
    p = p * t + jnp.float32(2.0000714765e-1)
    p = p * t + jnp.float32(-2.4999993993e-1)
    p = p * t + jnp.float32(3.3333331174e-1)
    t2 = t * t
    y = t2 * (t * p - 0.5)
    return t + y + e * jnp.float32(_LN2)


@functools.partial(
    pl.kernel,
    mesh=plsc.VectorSubcoreMesh(core_axis_name="c", subcore_axis_name="s"),
    out_type=jax.ShapeDtypeStruct((N,), jnp.float32),
    compiler_params=pltpu.CompilerParams(needs_layout_passes=False, disable_bounds_checks=True, disable_semaphore_checks=True, skip_device_barrier=True),
    scratch_types=[
        pltpu.VMEM((RPW,), jnp.int32),
        pltpu.VMEM((RPW,), jnp.int32),
        pltpu.VMEM((BLK, 8, EXTRA), jnp.float32),
        pltpu.VMEM((RPW, 8, 128), jnp.float32),
        pltpu.VMEM((RPW,), jnp.float32),
        pltpu.SemaphoreType.DMA,
    ],
)
def _loss_kernel(scores_hbm, align_hbm, target_hbm, out_hbm,
                 align_v, target_v, slab_a, blk_t, out_v, sem):
    wid = lax.axis_index("s") * NC + lax.axis_index("c")
    base = wid * RPW
    cp_a = pltpu.async_copy(align_hbm.at[pl.ds(base, RPW)], align_v, sem)
    cp_t = pltpu.async_copy(target_hbm.at[pl.ds(base, RPW)], target_v, sem)
    copies = []
    for b in range(BLK):
        copies.append(pltpu.async_copy(
            scores_hbm.at[pl.ds(pl.multiple_of(base + b * 8, 8), 8),
                          pl.ds(OFFSET, EXTRA)],
            slab_a.at[b], sem))
    cp_a.wait()
    cp_t.wait()
    lanes = lax.iota(jnp.int32, 16)
    t_copies = []
    for c in range(CH):
        tv = target_v[pl.ds(c * L, L)]
        for j in range(L):
            r = c * L + j
            t_scal = jnp.max(jnp.where(lanes == j, tv, 0))
            t_tile = pl.multiple_of((t_scal >> 7) << 7, 128)
            t_copies.append(pltpu.async_copy(
                scores_hbm.at[pl.ds(pl.multiple_of(base + (r & ~7), 8), 8),
                              pl.ds(t_tile, 128)],
                blk_t.at[r], sem))
    out_copies = []
    for c in range(CH):
        copies[2 * c].wait()
        copies[2 * c + 1].wait()
        for cp in t_copies[c * L:(c + 1) * L]:
            cp.wait()
        av = align_v[pl.ds(c * L, L)]
        tv = target_v[pl.ds(c * L, L)]
        r_vec = c * L + lanes
        sub = r_vec & 7
        a_val = plsc.load_gather(slab_a, [r_vec >> 3, sub, av])
        t_val = plsc.load_gather(blk_t, [r_vec, sub, tv & 127])
        zero = jnp.zeros((L,), jnp.float32)
        a_unk = av == UNK
        t_unk = tv == UNK
        out = jnp.where(a_unk, zero, a_val) + jnp.float32(EPS)
        out = out + jnp.where(t_unk, zero, t_val)
        out = out + jnp.where(a_unk & t_unk, t_val, zero)
        loss = -_vlog(out)
        loss = jnp.where(tv == IGNORE_INDEX, zero, loss)
        out_v[pl.ds(c * L, L)] = loss
        out_copies.append(pltpu.async_copy(
            out_v.at[pl.ds(c * L, L)],
            out_hbm.at[pl.ds(base + c * L, L)], sem))
    for cp in out_copies:
        cp.wait()


def kernel(scores, align, target):
    return _loss_kernel(scores,
                        align.astype(jnp.int32),
                        target.astype(jnp.int32))


# speedup vs baseline: 1.2496x; 1.0144x over previous
import functools

import jax
import jax.numpy as jnp
from jax import lax
from jax.experimental import pallas as pl
from jax.experimental.pallas import tpu as pltpu
from jax.experimental.pallas import tpu_sc as plsc

N = 2048
ROW = 32512
OFFSET = 32000
EXTRA = ROW - OFFSET
EPS = 1e-20
UNK = 0
IGNORE_INDEX = -100

NC = 2
NS = 16
NW = NC * NS
RPW = N // NW
L = 16
CH = RPW // L
BLK = RPW // 8

_LN2 = 0.6931471805599453
_SQRTHF = 0.70710678118654752440


def _vlog(x):
    bits = lax.bitcast_convert_type(x, jnp.int32)
    e = lax.shift_right_arithmetic(bits, 23) - 126
    m_bits = (bits & 0x007FFFFF) | 0x3F000000
    m = lax.bitcast_convert_type(m_bits, jnp.float32)
    small = m < _SQRTHF
    e = jnp.where(small, e - 1, e).astype(jnp.float32)
    t = jnp.where(small, m + m, m) - 1.0
    p = jnp.float32(7.0376836292e-2)
    p = p * t + jnp.float32(-1.1514610310e-1)
    p = p * t + jnp.float32(1.1676998740e-1)
    p = p * t + jnp.float32(-1.2420140846e-1)
    p = p * t + jnp.float32(1.4249322787e-1)
    p = p * t + jnp.float32(-1.6668057665e-1)
    p = p * t + jnp.float32(2.0000714765e-1)
    p = p * t + jnp.float32(-2.4999993993e-1)
    p = p * t + jnp.float32(3.3333331174e-1)
    t2 = t * t
    y = t2 * (t * p - 0.5)
    return t + y + e * jnp.float32(_LN2)


@functools.partial(
    pl.kernel,
    mesh=plsc.VectorSubcoreMesh(core_axis_name="c", subcore_axis_name="s"),
    out_type=jax.ShapeDtypeStruct((N,), jnp.float32),
    compiler_params=pltpu.CompilerParams(needs_layout_passes=False, disable_bounds_checks=True, disable_semaphore_checks=True, skip_device_barrier=True),
    scratch_types=[
        pltpu.VMEM((RPW,), jnp.int32),
        pltpu.VMEM((RPW,), jnp.int32),
        pltpu.VMEM((BLK, 8, EXTRA), jnp.float32),
        pltpu.VMEM((RPW, 8, 128), jnp.float32),
        pltpu.VMEM((RPW,), jnp.float32),
        pltpu.SemaphoreType.DMA,
    ],
)
def _loss_kernel(scores_hbm, align_hbm, target_hbm, out_hbm,
                 align_v, target_v, slab_a, blk_t, out_v, sem):
    wid = lax.axis_index("s") * NC + lax.axis_index("c")
    base = wid * RPW
    cp_a = pltpu.async_copy(align_hbm.at[pl.ds(base, RPW)], align_v, sem)
    cp_t = pltpu.async_copy(target_hbm.at[pl.ds(base, RPW)], target_v, sem)
    copies = []
    for b in range(BLK):
        copies.append(pltpu.async_copy(
            scores_hbm.at[pl.ds(pl.multiple_of(base + b * 8, 8), 8),
                          pl.ds(OFFSET, EXTRA)],
            slab_a.at[b], sem))
    cp_a.wait()
    cp_t.wait()
    lanes = lax.iota(jnp.int32, 16)
    for c in range(CH):
        tv = target_v[pl.ds(c * L, L)]
        tidx = lax.shift_right_logical(tv, 7)
        q0 = jnp.take(tidx, (lanes * 4) & 15, axis=0)
        q1 = jnp.take(tidx, (lanes * 4 + 1) & 15, axis=0)
        q2 = jnp.take(tidx, (lanes * 4 + 2) & 15, axis=0)
        q3 = jnp.take(tidx, (lanes * 4 + 3) & 15, axis=0)
        packed = q0 | (q1 << 8) | (q2 << 16) | (q3 << 24)
        for k in range(4):
            p_scal = jnp.min(jnp.where(lanes == k, packed, 0x7FFFFFFF))
            for m in range(4):
                r = c * L + 4 * k + m
                t_tile = pl.multiple_of(((p_scal >> (8 * m)) & 0xFF) << 7, 128)
                copies.append(pltpu.async_copy(
                    scores_hbm.at[pl.ds(pl.multiple_of(base + (r & ~7), 8), 8),
                                  pl.ds(t_tile, 128)],
                    blk_t.at[r], sem))
    for cp in copies:
        cp.wait()
    for c in range(CH):
        av = align_v[pl.ds(c * L, L)]
        tv = target_v[pl.ds(c * L, L)]
        r_vec = c * L + lanes
        sub = r_vec & 7
        a_val = plsc.load_gather(slab_a, [r_vec >> 3, sub, av])
        t_val = plsc.load_gather(blk_t, [r_vec, sub, tv & 127])
        zero = jnp.zeros((L,), jnp.float32)
        a_unk = av == UNK
        t_unk = tv == UNK
        out = jnp.where(a_unk, zero, a_val) + jnp.float32(EPS)
        out = out + jnp.where(t_unk & ~a_unk, zero, t_val)
        loss = -_vlog(out)
        loss = jnp.where(tv == IGNORE_INDEX, zero, loss)
        out_v[pl.ds(c * L, L)] = loss
    pltpu.sync_copy(out_v, out_hbm.at[pl.ds(base, RPW)])


def kernel(scores, align, target):
    return _loss_kernel(scores,
                        align.astype(jnp.int32),
                        target.astype(jnp.int32))


# speedup vs baseline: 1.2664x; 1.0134x over previous
import functools

import jax
import jax.numpy as jnp
from jax import lax
from jax.experimental import pallas as pl
from jax.experimental.pallas import tpu as pltpu
from jax.experimental.pallas import tpu_sc as plsc

N = 2048
ROW = 32512
OFFSET = 32000
EXTRA = ROW - OFFSET
EPS = 1e-20
UNK = 0
IGNORE_INDEX = -100

NC = 2
NS = 16
NW = NC * NS
RPW = N // NW
L = 16
CH = RPW // L
BLK = RPW // 8

_LN2 = 0.6931471805599453
_SQRTHF = 0.70710678118654752440


def _vlog(x):
    bits = lax.bitcast_convert_type(x, jnp.int32)
    e = lax.shift_right_arithmetic(bits, 23) - 126
    m_bits = (bits & 0x007FFFFF) | 0x3F000000
    m = lax.bitcast_convert_type(m_bits, jnp.float32)
    small = m < _SQRTHF
    e = jnp.where(small, e - 1, e).astype(jnp.float32)
    t = jnp.where(small, m + m, m) - 1.0
    p = jnp.float32(7.0376836292e-2)
    p = p * t + jnp.float32(-1.1514610310e-1)
    p = p * t + jnp.float32(1.1676998740e-1)
    p = p * t + jnp.float32(-1.2420140846e-1)
    p = p * t + jnp.float32(1.4249322787e-1)
    p = p * t + jnp.float32(-1.6668057665e-1)
    p = p * t + jnp.float32(2.0000714765e-1)
    p = p * t + jnp.float32(-2.4999993993e-1)
    p = p * t + jnp.float32(3.3333331174e-1)
    t2 = t * t
    y = t2 * (t * p - 0.5)
    return t + y + e * jnp.float32(_LN2)


@functools.partial(
    pl.kernel,
    mesh=plsc.VectorSubcoreMesh(core_axis_name="c", subcore_axis_name="s"),
    out_type=jax.ShapeDtypeStruct((N,), jnp.float32),
    compiler_params=pltpu.CompilerParams(needs_layout_passes=False, disable_bounds_checks=True, disable_semaphore_checks=True, skip_device_barrier=True),
    scratch_types=[
        pltpu.VMEM((RPW,), jnp.int32),
        pltpu.VMEM((RPW,), jnp.int32),
        pltpu.VMEM((RPW, EXTRA), jnp.float32),
        pltpu.VMEM((RPW, 8, 128), jnp.float32),
        pltpu.VMEM((RPW,), jnp.float32),
        pltpu.SemaphoreType.DMA,
    ],
)
def _loss_kernel(scores_hbm, align_hbm, target_hbm, out_hbm,
                 align_v, target_v, slab_a, blk_t, out_v, sem):
    wid = lax.axis_index("s") * NC + lax.axis_index("c")
    base = wid * RPW
    cp_a = pltpu.async_copy(align_hbm.at[pl.ds(base, RPW)], align_v, sem)
    cp_t = pltpu.async_copy(target_hbm.at[pl.ds(base, RPW)], target_v, sem)
    copies = [pltpu.async_copy(
        scores_hbm.at[pl.ds(pl.multiple_of(base, 8), RPW), pl.ds(OFFSET, EXTRA)],
        slab_a, sem)]
    cp_t.wait()
    lanes = lax.iota(jnp.int32, 16)
    for c in range(CH):
        tv = target_v[pl.ds(c * L, L)]
        tidx = lax.shift_right_logical(tv, 7)
        q0 = jnp.take(tidx, (lanes * 4) & 15, axis=0)
        q1 = jnp.take(tidx, (lanes * 4 + 1) & 15, axis=0)
        q2 = jnp.take(tidx, (lanes * 4 + 2) & 15, axis=0)
        q3 = jnp.take(tidx, (lanes * 4 + 3) & 15, axis=0)
        packed = q0 | (q1 << 8) | (q2 << 16) | (q3 << 24)
        for k in range(4):
            p_scal = jnp.min(jnp.where(lanes == k, packed, 0x7FFFFFFF))
            for m in range(4):
                r = c * L + 4 * k + m
                t_tile = pl.multiple_of(((p_scal >> (8 * m)) & 0xFF) << 7, 128)
                copies.append(pltpu.async_copy(
                    scores_hbm.at[pl.ds(pl.multiple_of(base + (r & ~7), 8), 8),
                                  pl.ds(t_tile, 128)],
                    blk_t.at[r], sem))
    cp_a.wait()
    for cp in copies:
        cp.wait()
    for c in range(CH):
        av = align_v[pl.ds(c * L, L)]
        tv = target_v[pl.ds(c * L, L)]
        r_vec = c * L + lanes
        sub = r_vec & 7
        a_val = plsc.load_gather(slab_a, [r_vec, av])
        t_val = plsc.load_gather(blk_t, [r_vec, sub, tv & 127])
        zero = jnp.zeros((L,), jnp.float32)
        a_unk = av == UNK
        t_unk = tv == UNK
        out = jnp.where(a_unk, zero, a_val) + jnp.float32(EPS)
        out = out + jnp.where(t_unk & ~a_unk, zero, t_val)
        loss = -_vlog(out)
        loss = jnp.where(tv == IGNORE_INDEX, zero, loss)
        out_v[pl.ds(c * L, L)] = loss
    pltpu.sync_copy(out_v, out_hbm.at[pl.ds(base, RPW)])


def kernel(scores, align, target):
    return _loss_kernel(scores,
                        align.astype(jnp.int32),
                        target.astype(jnp.int32))
